# baseline (speedup 1.0000x reference)
import dataclasses
import functools

import jax
import jax.numpy as jnp
from jax import lax
from jax.experimental import pallas as pl
from jax.experimental.pallas import tpu as pltpu
from jax.experimental.pallas import tpu_sc as plsc

GS = 128
NLEV = 16
L = 16
BLOCK = 16384
GPI = 4
INV_MAXQ2 = 2.0 / 15.0


def kernel(x, lookup_values):
    shape = x.shape
    n = x.size
    x1 = x.reshape(n)
    mesh = plsc.VectorSubcoreMesh(core_axis_name="c", subcore_axis_name="s")
    cp = pltpu.CompilerParams()
    if "needs_layout_passes" in pltpu.CompilerParams.__dataclass_fields__:
        cp = dataclasses.replace(cp, needs_layout_passes=False)

    @functools.partial(
        pl.kernel,
        mesh=mesh,
        out_type=jax.ShapeDtypeStruct((n,), jnp.float32),
        scratch_types=[pltpu.VMEM((NLEV,), jnp.float32)],
        compiler_params=cp,
    )
    def run(x_hbm, lut_hbm, o_hbm, lut_vmem):
        pltpu.sync_copy(lut_hbm, lut_vmem)
        lutv = lut_vmem[...]

        def take(v, idx):
            return v.at[idx].get(mode="promise_in_bounds")

        lane = lax.iota(jnp.int32, L)
        shifted = take(lutv, jnp.minimum(lane + 1, NLEV - 1))
        midsv = (lutv + shifted) * 0.5

        i7 = jnp.full((L,), 7, jnp.int32)
        s8 = jnp.full((L,), 8, jnp.int32)
        s4 = jnp.full((L,), 4, jnp.int32)
        s2 = jnp.full((L,), 2, jnp.int32)
        s1 = jnp.full((L,), 1, jnp.int32)
        s0 = jnp.zeros((L,), jnp.int32)

        def search(q, msv, valv):
            lo = jnp.where(q > take(msv, i7), s8, s0)
            lo = lo + jnp.where(q > take(msv, lo + 3), s4, s0)
            lo = lo + jnp.where(q > take(msv, lo + 1), s2, s0)
            lo = lo + jnp.where(q > take(msv, lo), s1, s0)
            return take(valv, lo)

        zv = search(jnp.full((L,), 8.0, jnp.float32), midsv, lutv)
        mz = midsv - zv
        oz = lutv - zv

        def body(x_vmem, o_vmem):
            @pl.loop(0, BLOCK // GS, step=GPI)
            def _(g0):
                for gg in range(GPI):
                    base = (g0 + gg) * GS
                    xs = [
                        x_vmem[pl.ds(base + j * L, L)] for j in range(GS // L)
                    ]
                    avs = [jnp.abs(v) for v in xs]
                    while len(avs) > 1:
                        avs = [
                            jnp.maximum(avs[k], avs[k + 1])
                            for k in range(0, len(avs) - 1, 2)
                        ] + ([avs[-1]] if len(avs) % 2 else [])
                    amax = jnp.max(avs[0])
                    amaxv = jnp.full((L,), amax, jnp.float32)
                    scale = jnp.where(
                        amaxv == 0.0, INV_MAXQ2, amaxv * INV_MAXQ2
                    )
                    msv = mz * scale
                    outv = oz * scale
                    for j in range(GS // L):
                        o_vmem[pl.ds(base + j * L, L)] = search(
                            xs[j], msv, outv
                        )

        pltpu.emit_pipeline(
            body,
            grid=(n // BLOCK,),
            in_specs=[pl.BlockSpec((BLOCK,), lambda i: (i,))],
            out_specs=[pl.BlockSpec((BLOCK,), lambda i: (i,))],
            core_axis_name=("c", "s"),
            dimension_semantics=(pltpu.PARALLEL,),
        )(x_hbm, o_hbm)

    return run(x1, lookup_values).reshape(shape)


# speedup vs baseline: 1.0672x; 1.0672x over previous
import dataclasses
import functools

import jax
import jax.numpy as jnp
from jax import lax
from jax.experimental import pallas as pl
from jax.experimental.pallas import tpu as pltpu
from jax.experimental.pallas import tpu_sc as plsc

GS = 128
NLEV = 16
L = 16
NC = 2
NS = 16
NW = NC * NS
CH = 16384
NB = 2
GPI = 2
INV_MAXQ2 = 2.0 / 15.0


def kernel(x, lookup_values):
    shape = x.shape
    n = x.size
    x1 = x.reshape(n)
    per_w = n // NW
    nch = per_w // CH
    mesh = plsc.VectorSubcoreMesh(core_axis_name="c", subcore_axis_name="s")
    cp = pltpu.CompilerParams()
    if "needs_layout_passes" in pltpu.CompilerParams.__dataclass_fields__:
        cp = dataclasses.replace(cp, needs_layout_passes=False)

    @functools.partial(
        pl.kernel,
        mesh=mesh,
        out_type=jax.ShapeDtypeStruct((n,), jnp.float32),
        scratch_types=[
            pltpu.VMEM((NLEV,), jnp.float32),
            pltpu.VMEM((NB, CH), jnp.float32),
            pltpu.VMEM((NB, CH), jnp.float32),
            pltpu.SemaphoreType.DMA((NB,)),
            pltpu.SemaphoreType.DMA((NB,)),
        ],
        compiler_params=cp,
    )
    def run(x_hbm, lut_hbm, o_hbm, lut_vmem, ibuf, obuf, isem, osem):
        pltpu.sync_copy(lut_hbm, lut_vmem)
        lutv = lut_vmem[...]

        def take(v, idx):
            return v.at[idx].get(mode="promise_in_bounds")

        lane = lax.iota(jnp.int32, L)
        shifted = take(lutv, jnp.minimum(lane + 1, NLEV - 1))
        midsv = (lutv + shifted) * 0.5

        i7 = jnp.full((L,), 7, jnp.int32)
        s8 = jnp.full((L,), 8, jnp.int32)
        s4 = jnp.full((L,), 4, jnp.int32)
        s2 = jnp.full((L,), 2, jnp.int32)
        s1 = jnp.full((L,), 1, jnp.int32)
        s0 = jnp.zeros((L,), jnp.int32)

        def search(q, msv, valv):
            lo = jnp.where(q > take(msv, i7), s8, s0)
            lo = lo + jnp.where(q > take(msv, lo + 3), s4, s0)
            lo = lo + jnp.where(q > take(msv, lo + 1), s2, s0)
            lo = lo + jnp.where(q > take(msv, lo), s1, s0)
            return take(valv, lo)

        zv = search(jnp.full((L,), 8.0, jnp.float32), midsv, lutv)
        mz = midsv - zv
        oz = lutv - zv

        wid = lax.axis_index("s") * NC + lax.axis_index("c")
        base = wid * per_w

        def in_cp(i, b):
            return pltpu.make_async_copy(
                x_hbm.at[pl.ds(base + i * CH, CH)], ibuf.at[b], isem.at[b]
            )

        def out_cp(i, b):
            return pltpu.make_async_copy(
                obuf.at[b], o_hbm.at[pl.ds(base + i * CH, CH)], osem.at[b]
            )

        for b in range(NB):
            in_cp(b, b).start()

        @pl.loop(0, nch, step=NB)
        def _(i0):
            for b in range(NB):
                i = i0 + b
                in_cp(i, b).wait()

                @pl.loop(0, CH // GS, step=GPI)
                def _(g0):
                    for gg in range(GPI):
                        gbase = (g0 + gg) * GS
                        xs = [
                            ibuf[b, pl.ds(gbase + j * L, L)]
                            for j in range(GS // L)
                        ]
                        avs = [jnp.abs(v) for v in xs]
                        while len(avs) > 1:
                            avs = [
                                jnp.maximum(avs[k], avs[k + 1])
                                for k in range(0, len(avs) - 1, 2)
                            ] + ([avs[-1]] if len(avs) % 2 else [])
                        amax = jnp.max(avs[0])
                        amaxv = jnp.full((L,), amax, jnp.float32)
                        scale = jnp.where(
                            amaxv == 0.0, INV_MAXQ2, amaxv * INV_MAXQ2
                        )
                        msv = mz * scale
                        outv = oz * scale
                        for j in range(GS // L):
                            obuf[b, pl.ds(gbase + j * L, L)] = search(
                                xs[j], msv, outv
                            )

                @pl.when(i + NB < nch)
                def _():
                    in_cp(i + NB, b).start()

                @pl.when(i >= NB)
                def _():
                    out_cp(i - NB, b).wait()

                out_cp(i, b).start()

        for b in range(NB):
            out_cp(nch - NB + b, b).wait()

    return run(x1, lookup_values).reshape(shape)


# speedup vs baseline: 1.1758x; 1.1017x over previous
import dataclasses
import functools

import jax
import jax.numpy as jnp
from jax import lax
from jax.experimental import pallas as pl
from jax.experimental.pallas import tpu as pltpu
from jax.experimental.pallas import tpu_sc as plsc

GS = 128
NLEV = 16
L = 16
NC = 2
NS = 16
NW = NC * NS
CH = 8192
NB = 2
GPI = 2
INV_MAXQ2 = 2.0 / 15.0


def kernel(x, lookup_values):
    shape = x.shape
    n = x.size
    x1 = x.reshape(n)
    per_w = n // NW
    nch = per_w // CH
    mesh = plsc.VectorSubcoreMesh(core_axis_name="c", subcore_axis_name="s")
    cp = pltpu.CompilerParams()
    if "needs_layout_passes" in pltpu.CompilerParams.__dataclass_fields__:
        cp = dataclasses.replace(cp, needs_layout_passes=False)

    @functools.partial(
        pl.kernel,
        mesh=mesh,
        out_type=jax.ShapeDtypeStruct((n,), jnp.float32),
        scratch_types=[
            pltpu.VMEM((NLEV,), jnp.float32),
            pltpu.VMEM((CH,), jnp.float32),
            pltpu.VMEM((CH,), jnp.float32),
            pltpu.VMEM((CH,), jnp.float32),
            pltpu.VMEM((CH,), jnp.float32),
            pltpu.SemaphoreType.DMA((NB,)),
            pltpu.SemaphoreType.DMA((NB,)),
        ],
        compiler_params=cp,
    )
    def run(x_hbm, lut_hbm, o_hbm, lut_vmem, ibuf0, ibuf1, obuf0, obuf1, isem, osem):
        ibufs = [ibuf0, ibuf1]
        obufs = [obuf0, obuf1]
        pltpu.sync_copy(lut_hbm, lut_vmem)
        lutv = lut_vmem[...]

        def take(v, idx):
            return v.at[idx].get(mode="promise_in_bounds")

        lane = lax.iota(jnp.int32, L)
        shifted = take(lutv, jnp.minimum(lane + 1, NLEV - 1))
        midsv = (lutv + shifted) * 0.5

        i7 = jnp.full((L,), 7, jnp.int32)
        s8 = jnp.full((L,), 8, jnp.int32)
        s4 = jnp.full((L,), 4, jnp.int32)
        s2 = jnp.full((L,), 2, jnp.int32)
        s1 = jnp.full((L,), 1, jnp.int32)
        s0 = jnp.zeros((L,), jnp.int32)

        def search(q, msv, valv):
            lo = jnp.where(q > take(msv, i7), s8, s0)
            lo = lo + jnp.where(q > take(msv, lo + 3), s4, s0)
            lo = lo + jnp.where(q > take(msv, lo + 1), s2, s0)
            lo = lo + jnp.where(q > take(msv, lo), s1, s0)
            return take(valv, lo)

        zv = search(jnp.full((L,), 8.0, jnp.float32), midsv, lutv)
        mz = midsv - zv
        oz = lutv - zv

        wid = lax.axis_index("s") * NC + lax.axis_index("c")
        base = wid * per_w

        def in_cp(i, b):
            return pltpu.make_async_copy(
                x_hbm.at[pl.ds(base + i * CH, CH)], ibufs[b], isem.at[b]
            )

        def out_cp(i, b):
            return pltpu.make_async_copy(
                obufs[b], o_hbm.at[pl.ds(base + i * CH, CH)], osem.at[b]
            )

        for b in range(NB):
            in_cp(b, b).start()

        @pl.loop(0, nch, step=NB)
        def _(i0):
            for b in range(NB):
                i = i0 + b
                in_cp(i, b).wait()

                @pl.when(i >= NB)
                def _():
                    out_cp(i - NB, b).wait()

                @pl.loop(0, CH // GS, step=GPI)
                def _(g0):
                    for gg in range(GPI):
                        gbase = (g0 + gg) * GS
                        xs = [
                            ibufs[b][pl.ds(gbase + j * L, L)]
                            for j in range(GS // L)
                        ]
                        avs = [jnp.abs(v) for v in xs]
                        while len(avs) > 1:
                            avs = [
                                jnp.maximum(avs[k], avs[k + 1])
                                for k in range(0, len(avs) - 1, 2)
                            ] + ([avs[-1]] if len(avs) % 2 else [])
                        amax = jnp.max(avs[0])
                        amaxv = jnp.full((L,), amax, jnp.float32)
                        scale = jnp.where(
                            amaxv == 0.0, INV_MAXQ2, amaxv * INV_MAXQ2
                        )
                        msv = mz * scale
                        outv = oz * scale
                        for j in range(GS // L):
                            obufs[b][pl.ds(gbase + j * L, L)] = search(
                                xs[j], msv, outv
                            )

                @pl.when(i + NB < nch)
                def _():
                    in_cp(i + NB, b).start()

                out_cp(i, b).start()

        for b in range(NB):
            out_cp(nch - NB + b, b).wait()

    return run(x1, lookup_values).reshape(shape)
